# Initial kernel scaffold; baseline (speedup 1.0000x reference)
import jax
import jax.numpy as jnp
from jax.experimental import pallas as pl


def kernel(featuremap, boxes, box_ind):
    raise NotImplementedError("write your pallas kernel here")


# speedup vs baseline: 99.2754x; 99.2754x over previous
import functools

import jax
import jax.numpy as jnp
from jax import lax
from jax.experimental import pallas as pl
from jax.experimental.pallas import tpu as pltpu
from jax.experimental.pallas import tpu_sc as plsc

CROP = 7

NC = 2
NS = 16
L = 16
NW = NC * NS


def _roialign_sc(n, c, w, m, bpt):
    nsg = bpt // L
    mesh = plsc.VectorSubcoreMesh(
        core_axis_name="c", subcore_axis_name="s", num_cores=NC,
        num_subcores=NS)

    @functools.partial(
        pl.kernel,
        out_type=jax.ShapeDtypeStruct((m, c * CROP), jnp.float32),
        mesh=mesh,
        compiler_params=pltpu.CompilerParams(needs_layout_passes=False),
        scratch_types=[
            pltpu.VMEM((bpt,), jnp.float32),
            pltpu.VMEM((bpt,), jnp.float32),
            pltpu.VMEM((bpt,), jnp.int32),
            pltpu.VMEM((2 * CROP * L,), jnp.int32),
            pltpu.VMEM((2 * CROP * L, c), jnp.float32),
            pltpu.VMEM((L * L,), jnp.float32),
            pltpu.VMEM((c * CROP,), jnp.float32),
            pltpu.SemaphoreType.DMA,
        ],
    )
    def kern(x1_hbm, x2_hbm, bi_hbm, fmt_hbm, out_hbm,
             x1c, x2c, bic, idxv, rows, wv, tbuf, sem):
        wid = lax.axis_index("s") * NC + lax.axis_index("c")
        base = wid * bpt
        pltpu.sync_copy(x1_hbm.at[pl.ds(base, bpt)], x1c)
        pltpu.sync_copy(x2_hbm.at[pl.ds(base, bpt)], x2c)
        pltpu.sync_copy(bi_hbm.at[pl.ds(base, bpt)], bic)

        lane = lax.iota(jnp.int32, L)
        lane7 = lane * CROP
        lane16 = lane * L
        wm1f = float(w - 1)

        def supergroup(sg, _):
            gbase = sg * L
            x1 = x1c[pl.ds(gbase, L)]
            x2 = x2c[pl.ds(gbase, L)]
            rowb = bic[pl.ds(gbase, L)] * w
            sp = (x2 - x1) / float(CROP)
            x1n = (x1 + sp * 0.5 - 0.5) / wm1f
            x2n = x1n + sp * float(CROP - 1) / wm1f
            step = (x2n - x1n) * wm1f / float(CROP - 1)
            xs0 = x1n * wm1f
            for j in range(CROP):
                xs = xs0 + float(j) * step
                x0i = xs.astype(jnp.int32)
                i0 = jnp.clip(x0i, 0, w - 1)
                idxv[pl.ds((2 * j) * L, L)] = rowb + i0
                idxv[pl.ds((2 * j + 1) * L, L)] = rowb + jnp.minimum(i0 + 1, w - 1)
                f = xs - x0i.astype(jnp.float32)
                vf = jnp.where((xs >= 0.0) & (xs <= wm1f), 1.0, 0.0)
                w1 = f * vf
                plsc.store_scatter(wv, [lane16 + j], vf - w1)
                plsc.store_scatter(wv, [lane16 + (8 + j)], w1)
            pltpu.async_copy(fmt_hbm.at[idxv], rows, sem).wait()

            def box(k, _):
                wk = wv[pl.ds(k * L, L)]
                for j in range(CROP):
                    a0 = wk[j]
                    a1 = wk[8 + j]
                    r0 = (2 * j) * L + k
                    r1 = r0 + L
                    for cc in range(c // L):
                        g0 = rows[r0, pl.ds(cc * L, L)]
                        g1 = rows[r1, pl.ds(cc * L, L)]
                        plsc.store_scatter(
                            tbuf, [lane7 + (cc * L * CROP + j)],
                            g0 * a0 + g1 * a1)
                mg = base + gbase + k

                @pl.when(mg < m)
                def _():
                    pltpu.sync_copy(tbuf, out_hbm.at[mg])

                return 0

            lax.fori_loop(0, L, box, 0)
            return 0

        lax.fori_loop(0, nsg, supergroup, 0)

    return kern


def kernel(featuremap, boxes, box_ind):
    n, c, w = featuremap.shape
    m = boxes.shape[0]
    bpt = -(-m // (NW * L)) * L
    m_pad = bpt * NW

    fm_t = jnp.transpose(featuremap, (0, 2, 1)).reshape(n * w, c)
    pad = m_pad - m
    x1 = jnp.concatenate([boxes[:, 0], jnp.zeros((pad,), jnp.float32)])
    x2 = jnp.concatenate([boxes[:, 1], jnp.zeros((pad,), jnp.float32)])
    bi = jnp.concatenate([box_ind, jnp.zeros((pad,), jnp.int32)])

    out = _roialign_sc(n, c, w, m, bpt)(x1, x2, bi, fm_t)
    return out.reshape(m, c, CROP)
